# baseline (speedup 1.0000x reference)
import functools

import jax
import jax.numpy as jnp
from jax import lax
from jax.experimental import pallas as pl
from jax.experimental.pallas import tpu as pltpu
from jax.experimental.pallas import tpu_sc as plsc

NLOC, NALL, NSEL = 50000, 60000, 32
NTYPES = 4
M1, M2 = 16, 4
RMIN, RMAX = 0.5, 6.0

NLOC_PAD = 51200
NPAIR = NLOC_PAD * NSEL
NWORK = 32
PW = NPAIR // NWORK
CHUNK = 1024
NCHUNK = PW // CHUNK
A_BLK = 512
P_ROWS = A_BLK * NSEL // 128
N_BLOCKS = NLOC_PAD // A_BLK


NBUF = 3
C0_CHUNKS = 50


def _sc_gather_body(coord_hbm, nlist_hbm, dx_hbm, dy_hbm, dz_hbm, *refs):
    bufs = [refs[5 * b:5 * b + 5] for b in range(NBUF)]
    sems = refs[5 * NBUF:]
    idx_sems = sems[0:NBUF]
    in_sems = sems[NBUF:2 * NBUF]
    out_sems = sems[2 * NBUF:3 * NBUF]
    cid = lax.axis_index("c")
    sid = lax.axis_index("s")
    lane = lax.iota(jnp.int32, 16)
    sub_chunks = 2 * NCHUNK
    c0_base = sid * sub_chunks * CHUNK
    my_base = c0_base + cid * (C0_CHUNKS * CHUNK)

    def chunk_base(k):
        return pl.multiple_of(my_base + k * CHUNK, 256)

    def start_idx(k):
        b = k % NBUF
        return pltpu.async_copy(nlist_hbm.at[pl.ds(chunk_base(k), CHUNK)],
                                bufs[b][0], idx_sems[b])

    def start_gather(k):
        b = k % NBUF
        return pltpu.async_copy(coord_hbm.at[bufs[b][0]], bufs[b][1],
                                in_sems[b])

    def start_outputs(k):
        b = k % NBUF
        _, _, dxv, dyv, dzv = bufs[b]
        base = chunk_base(k)
        return [pltpu.async_copy(dxv, dx_hbm.at[pl.ds(base, CHUNK)], out_sems[b]),
                pltpu.async_copy(dyv, dy_hbm.at[pl.ds(base, CHUNK)], out_sems[b]),
                pltpu.async_copy(dzv, dz_hbm.at[pl.ds(base, CHUNK)], out_sems[b])]

    def compute(k):
        b = k % NBUF
        _, rows, dxv, dyv, dzv = bufs[b]

        def body(t, _):
            p0 = t * 16
            pv = lane + p0
            sl = pl.ds(p0, 16)
            col0 = jnp.full((16,), 0, jnp.int32)
            col1 = jnp.full((16,), 1, jnp.int32)
            col2 = jnp.full((16,), 2, jnp.int32)
            dxv[sl] = plsc.load_gather(rows, [pv, col0])
            dyv[sl] = plsc.load_gather(rows, [pv, col1])
            dzv[sl] = plsc.load_gather(rows, [pv, col2])
            return _

        lax.fori_loop(0, CHUNK // 16, body, None)

    def pipeline(nchunks):
        start_idx(0).wait()
        gat = {0: start_gather(0)}
        idxp = {1: start_idx(1)} if nchunks > 1 else {}
        outs = {}
        for k in range(nchunks):
            if k + 1 < nchunks:
                idxp.pop(k + 1).wait()
                gat[k + 1] = start_gather(k + 1)
            if k + 2 < nchunks:
                idxp[k + 2] = start_idx(k + 2)
            gat.pop(k).wait()
            if k - NBUF + 1 in outs:
                for c in outs.pop(k - NBUF + 1):
                    c.wait()
            compute(k)
            outs[k] = start_outputs(k)
        for cs in outs.values():
            for c in cs:
                c.wait()

    @pl.when(cid == 0)
    def _():
        pipeline(C0_CHUNKS)

    @pl.when(cid == 1)
    def _():
        pipeline(2 * NCHUNK - C0_CHUNKS)


@functools.lru_cache(maxsize=1)
def _sc_gather_fn():
    bufset = [
        pltpu.VMEM((CHUNK,), jnp.int32),
        pltpu.VMEM((CHUNK, 8), jnp.float32),
        pltpu.VMEM((CHUNK,), jnp.float32),
        pltpu.VMEM((CHUNK,), jnp.float32),
        pltpu.VMEM((CHUNK,), jnp.float32),
    ]
    return pl.kernel(
        _sc_gather_body,
        out_type=(jax.ShapeDtypeStruct((NPAIR,), jnp.float32),) * 3,
        mesh=plsc.VectorSubcoreMesh(core_axis_name="c", subcore_axis_name="s"),
        compiler_params=pltpu.CompilerParams(needs_layout_passes=False,
                                             use_tc_tiling_on_sc=False),
        scratch_types=bufset * NBUF + [pltpu.SemaphoreType.DMA] * (3 * NBUF),
    )


def _iota2(shape, axis):
    return lax.broadcasted_iota(jnp.int32, shape, axis)


def _tc_body(dx_ref, dy_ref, dz_ref, atype_ref, xl_ref, yl_ref, zl_ref,
             w0_ref, b0_ref, w1_ref, b1_ref,
             fw0_ref, fb0_ref, fw1_ref, fb1_ref, fw2_ref,
             fb2_ref, bias_ref, out_ref):
    f32 = jnp.float32
    lgrp = _iota2((P_ROWS, 128), 1) // 32

    def _expand(ref):
        a = ref[...]
        xi = jnp.broadcast_to(a[:, 0:1], (P_ROWS, 128))
        for c in range(1, 4):
            xi = jnp.where(lgrp == c,
                           jnp.broadcast_to(a[:, c:c + 1], (P_ROWS, 128)), xi)
        return xi

    dx = dx_ref[...] - _expand(xl_ref)
    dy = dy_ref[...] - _expand(yl_ref)
    dz = dz_ref[...] - _expand(zl_ref)
    r2 = dx * dx + dy * dy + dz * dz + 1e-12
    r = jnp.sqrt(r2)
    rinv = 1.0 / r
    uu = jnp.clip((r - RMIN) / (RMAX - RMIN), 0.0, 1.0)
    uu2 = uu * uu
    sw = uu2 * uu * (-6.0 * uu2 + 15.0 * uu - 10.0) + 1.0
    sr = sw * rinv
    h = [jnp.tanh(sr * w0_ref[0, k] + b0_ref[k]) for k in range(8)]
    g = []
    for i in range(M1):
        acc = jnp.full_like(sr, b1_ref[i])
        for k in range(8):
            acc = acc + h[k] * w1_ref[k, i]
        g.append(jnp.tanh(acc))
    srr = sr * rinv
    R = [sr, srr * dx, srr * dy, srr * dz]

    rr = _iota2((4 * 128, M2 * 4), 0)
    cc = _iota2((4 * 128, M2 * 4), 1)
    bd512 = jnp.where((cc // 4 == rr // 128) & (cc % 4 == (rr % 128) // 32),
                      f32(1.0 / NSEL), f32(0.0))
    parts = []
    for i in range(M1):
        pi = jnp.concatenate([g[i] * R[j] for j in range(M2)], axis=1)
        parts.append(jax.lax.dot(pi, bd512, preferred_element_type=f32))
    grcat = jnp.concatenate(parts, axis=1)

    k4r = _iota2((M1 * M2, M1), 0)
    k4c = _iota2((M1 * M2, M1), 1)
    k4 = jnp.where(k4r // 4 == k4c, f32(1.0), f32(0.0))
    es = []
    for c in range(4):
        sr_ = _iota2((256, 64), 0)
        sc_ = _iota2((256, 64), 1)
        sel = jnp.where((sr_ % 4 == c) & (sr_ // 16 == sc_ // 4)
                        & ((sr_ % 16) // 4 == sc_ % 4), f32(1.0), f32(0.0))
        grc = jax.lax.dot(grcat, sel, preferred_element_type=f32)
        d = None
        for k in range(M2):
            br = _iota2((64, 64), 0)
            bc = _iota2((64, 64), 1)
            bk = jnp.where((br // 4 == k) & (br % 4 == bc % 4),
                           f32(1.0), f32(0.0))
            tk = jax.lax.dot(grc, bk, preferred_element_type=f32)
            dk = jax.lax.dot(grc * tk, k4, preferred_element_type=f32)
            skr = _iota2((M1, 64), 0)
            skc = _iota2((M1, 64), 1)
            s_k = jnp.where(skc == 4 * skr + k, f32(1.0), f32(0.0))
            dsc = jax.lax.dot(dk, s_k, preferred_element_type=f32)
            d = dsc if d is None else d + dsc
        hh = jnp.tanh(jax.lax.dot(d, fw0_ref[...],
                                  preferred_element_type=f32) + fb0_ref[...])
        h2 = jnp.tanh(jax.lax.dot(hh, fw1_ref[...],
                                  preferred_element_type=f32) + fb1_ref[...]) + hh
        ec = jax.lax.dot(h2, fw2_ref[...], preferred_element_type=f32)
        es.append(ec + fb2_ref[0])
    ecat = jnp.concatenate(es, axis=1)
    at = atype_ref[...]
    bias = jnp.zeros_like(ecat)
    for t in range(NTYPES):
        bias = jnp.where(at == t, bias_ref[t], bias)
    out_ref[...] = ecat + bias


def _tc_compute(dx, dy, dz, atype4, xl4, yl4, zl4, w0, b0, w1, b1,
                fw0, fb0, fw1, fb1, fw2, fb2, bias):
    rows_all = NPAIR // 128
    vspec = lambda: pl.BlockSpec((P_ROWS, 128), lambda b: (b, 0))
    aspec = lambda: pl.BlockSpec((P_ROWS, 4), lambda b: (b, 0))
    smem = lambda s: pl.BlockSpec(memory_space=pltpu.SMEM)
    full = lambda s: pl.BlockSpec(s, lambda b: tuple(0 for _ in s))
    return pl.pallas_call(
        _tc_body,
        grid=(N_BLOCKS,),
        in_specs=[vspec(), vspec(), vspec(), aspec(),
                  aspec(), aspec(), aspec(),
                  smem((1, 8)), smem((8,)), smem((8, 16)), smem((16,)),
                  full((64, 32)), full((1, 32)), full((32, 32)),
                  full((1, 32)), full((32, 1)),
                  smem((1,)), smem((4,))],
        out_specs=pl.BlockSpec((P_ROWS, 4), lambda b: (b, 0)),
        out_shape=jax.ShapeDtypeStruct((NLOC_PAD // 4, 4), jnp.float32),
    )(dx.reshape(rows_all, 128), dy.reshape(rows_all, 128),
      dz.reshape(rows_all, 128), atype4, xl4, yl4, zl4,
      w0, b0, w1, b1, fw0, fb0.reshape(1, 32), fw1, fb1.reshape(1, 32),
      fw2, fb2, bias)


@jax.jit
def kernel(extended_coord, extended_atype, nlist, w0, b0, w1, b1,
           fw0, fb0, fw1, fb1, fw2, fb2, bias_atom_e):
    coord4 = jnp.pad(extended_coord[0].astype(jnp.float32),
                     ((0, 0), (0, 5)))
    nflat = nlist[0].astype(jnp.int32).reshape(NLOC * NSEL)
    nflat = jnp.pad(nflat, (0, NPAIR - NLOC * NSEL))
    dx, dy, dz = _sc_gather_fn()(coord4, nflat)
    cl = extended_coord[0, :NLOC_PAD].astype(jnp.float32)
    xl4 = cl[:, 0].reshape(NLOC_PAD // 4, 4)
    yl4 = cl[:, 1].reshape(NLOC_PAD // 4, 4)
    zl4 = cl[:, 2].reshape(NLOC_PAD // 4, 4)
    atype4 = jnp.pad(extended_atype[0, :NLOC].astype(jnp.int32),
                     (0, NLOC_PAD - NLOC)).reshape(NLOC_PAD // 4, 4)
    e4 = _tc_compute(dx, dy, dz, atype4, xl4, yl4, zl4, w0, b0, w1, b1,
                     fw0, fb0, fw1, fb1, fw2, fb2, bias_atom_e)
    return e4.reshape(NLOC_PAD)[:NLOC].reshape(1, NLOC)


# speedup vs baseline: 1.4753x; 1.4753x over previous
import functools

import jax
import jax.numpy as jnp
from jax import lax
from jax.experimental import pallas as pl
from jax.experimental.pallas import tpu as pltpu
from jax.experimental.pallas import tpu_sc as plsc

NLOC, NALL, NSEL = 50000, 60000, 32
NTYPES = 4
M1, M2 = 16, 4
RMIN, RMAX = 0.5, 6.0

NLOC_PAD = 51200
NPAIR = NLOC_PAD * NSEL
NWORK = 32
PW = NPAIR // NWORK
CHUNK = 1024
NCHUNK = PW // CHUNK
A_BLK = 2048
P_ROWS = A_BLK * NSEL // 128
N_BLOCKS = NLOC_PAD // A_BLK


NBUF = 3
C0_CHUNKS = 50


def _sc_gather_body(coord_hbm, nlist_hbm, dx_hbm, dy_hbm, dz_hbm, *refs):
    bufs = [refs[5 * b:5 * b + 5] for b in range(NBUF)]
    sems = refs[5 * NBUF:]
    idx_sems = sems[0:NBUF]
    in_sems = sems[NBUF:2 * NBUF]
    out_sems = sems[2 * NBUF:3 * NBUF]
    cid = lax.axis_index("c")
    sid = lax.axis_index("s")
    lane = lax.iota(jnp.int32, 16)
    sub_chunks = 2 * NCHUNK
    c0_base = sid * sub_chunks * CHUNK
    my_base = c0_base + cid * (C0_CHUNKS * CHUNK)

    def chunk_base(k):
        return pl.multiple_of(my_base + k * CHUNK, 256)

    def start_idx(k):
        b = k % NBUF
        return pltpu.async_copy(nlist_hbm.at[pl.ds(chunk_base(k), CHUNK)],
                                bufs[b][0], idx_sems[b])

    def start_gather(k):
        b = k % NBUF
        return pltpu.async_copy(coord_hbm.at[bufs[b][0]], bufs[b][1],
                                in_sems[b])

    def start_outputs(k):
        b = k % NBUF
        _, _, dxv, dyv, dzv = bufs[b]
        base = chunk_base(k)
        return [pltpu.async_copy(dxv, dx_hbm.at[pl.ds(base, CHUNK)], out_sems[b]),
                pltpu.async_copy(dyv, dy_hbm.at[pl.ds(base, CHUNK)], out_sems[b]),
                pltpu.async_copy(dzv, dz_hbm.at[pl.ds(base, CHUNK)], out_sems[b])]

    def compute(k):
        b = k % NBUF
        _, rows, dxv, dyv, dzv = bufs[b]

        def body(t, _):
            p0 = t * 16
            pv = lane + p0
            sl = pl.ds(p0, 16)
            col0 = jnp.full((16,), 0, jnp.int32)
            col1 = jnp.full((16,), 1, jnp.int32)
            col2 = jnp.full((16,), 2, jnp.int32)
            dxv[sl] = plsc.load_gather(rows, [pv, col0])
            dyv[sl] = plsc.load_gather(rows, [pv, col1])
            dzv[sl] = plsc.load_gather(rows, [pv, col2])
            return _

        lax.fori_loop(0, CHUNK // 16, body, None)

    def pipeline(nchunks):
        start_idx(0).wait()
        gat = {0: start_gather(0)}
        idxp = {1: start_idx(1)} if nchunks > 1 else {}
        outs = {}
        for k in range(nchunks):
            if k + 1 < nchunks:
                idxp.pop(k + 1).wait()
                gat[k + 1] = start_gather(k + 1)
            if k + 2 < nchunks:
                idxp[k + 2] = start_idx(k + 2)
            gat.pop(k).wait()
            if k - NBUF + 1 in outs:
                for c in outs.pop(k - NBUF + 1):
                    c.wait()
            compute(k)
            outs[k] = start_outputs(k)
        for cs in outs.values():
            for c in cs:
                c.wait()

    @pl.when(cid == 0)
    def _():
        pipeline(C0_CHUNKS)

    @pl.when(cid == 1)
    def _():
        pipeline(2 * NCHUNK - C0_CHUNKS)


@functools.lru_cache(maxsize=1)
def _sc_gather_fn():
    bufset = [
        pltpu.VMEM((CHUNK,), jnp.int32),
        pltpu.VMEM((CHUNK, 8), jnp.float32),
        pltpu.VMEM((CHUNK,), jnp.float32),
        pltpu.VMEM((CHUNK,), jnp.float32),
        pltpu.VMEM((CHUNK,), jnp.float32),
    ]
    return pl.kernel(
        _sc_gather_body,
        out_type=(jax.ShapeDtypeStruct((NPAIR,), jnp.float32),) * 3,
        mesh=plsc.VectorSubcoreMesh(core_axis_name="c", subcore_axis_name="s"),
        compiler_params=pltpu.CompilerParams(needs_layout_passes=False,
                                             use_tc_tiling_on_sc=False),
        scratch_types=bufset * NBUF + [pltpu.SemaphoreType.DMA] * (3 * NBUF),
    )


def _iota2(shape, axis):
    return lax.broadcasted_iota(jnp.int32, shape, axis)


def _tc_body(dx_ref, dy_ref, dz_ref, atype_ref, xl_ref, yl_ref, zl_ref,
             w0_ref, b0_ref, w1_ref, b1_ref,
             fw0_ref, fb0_ref, fw1_ref, fb1_ref, fw2_ref,
             fb2_ref, bias_ref, out_ref):
    f32 = jnp.float32
    lgrp = _iota2((P_ROWS, 128), 1) // 32

    def _expand(ref):
        a = ref[...]
        xi = jnp.broadcast_to(a[:, 0:1], (P_ROWS, 128))
        for c in range(1, 4):
            xi = jnp.where(lgrp == c,
                           jnp.broadcast_to(a[:, c:c + 1], (P_ROWS, 128)), xi)
        return xi

    dx = dx_ref[...] - _expand(xl_ref)
    dy = dy_ref[...] - _expand(yl_ref)
    dz = dz_ref[...] - _expand(zl_ref)
    r2 = dx * dx + dy * dy + dz * dz + 1e-12
    r = jnp.sqrt(r2)
    rinv = 1.0 / r
    uu = jnp.clip((r - RMIN) / (RMAX - RMIN), 0.0, 1.0)
    uu2 = uu * uu
    sw = uu2 * uu * (-6.0 * uu2 + 15.0 * uu - 10.0) + 1.0
    sr = sw * rinv
    h = [jnp.tanh(sr * w0_ref[0, k] + b0_ref[k]) for k in range(8)]
    g = []
    for i in range(M1):
        acc = jnp.full_like(sr, b1_ref[i])
        for k in range(8):
            acc = acc + h[k] * w1_ref[k, i]
        g.append(jnp.tanh(acc))
    srr = sr * rinv
    R = [sr, srr * dx, srr * dy, srr * dz]

    rr = _iota2((4 * 128, M2 * 4), 0)
    cc = _iota2((4 * 128, M2 * 4), 1)
    bd512 = jnp.where((cc // 4 == rr // 128) & (cc % 4 == (rr % 128) // 32),
                      f32(1.0 / NSEL), f32(0.0))
    parts = []
    for i in range(M1):
        pi = jnp.concatenate([g[i] * R[j] for j in range(M2)], axis=1)
        parts.append(jax.lax.dot(pi, bd512, preferred_element_type=f32))
    grcat = jnp.concatenate(parts, axis=1)

    k4r = _iota2((M1 * M2, M1), 0)
    k4c = _iota2((M1 * M2, M1), 1)
    k4 = jnp.where(k4r // 4 == k4c, f32(1.0), f32(0.0))
    es = []
    for c in range(4):
        sr_ = _iota2((256, 64), 0)
        sc_ = _iota2((256, 64), 1)
        sel = jnp.where((sr_ % 4 == c) & (sr_ // 16 == sc_ // 4)
                        & ((sr_ % 16) // 4 == sc_ % 4), f32(1.0), f32(0.0))
        grc = jax.lax.dot(grcat, sel, preferred_element_type=f32)
        d = None
        for k in range(M2):
            br = _iota2((64, 64), 0)
            bc = _iota2((64, 64), 1)
            bk = jnp.where((br // 4 == k) & (br % 4 == bc % 4),
                           f32(1.0), f32(0.0))
            tk = jax.lax.dot(grc, bk, preferred_element_type=f32)
            dk = jax.lax.dot(grc * tk, k4, preferred_element_type=f32)
            skr = _iota2((M1, 64), 0)
            skc = _iota2((M1, 64), 1)
            s_k = jnp.where(skc == 4 * skr + k, f32(1.0), f32(0.0))
            dsc = jax.lax.dot(dk, s_k, preferred_element_type=f32)
            d = dsc if d is None else d + dsc
        hh = jnp.tanh(jax.lax.dot(d, fw0_ref[...],
                                  preferred_element_type=f32) + fb0_ref[...])
        h2 = jnp.tanh(jax.lax.dot(hh, fw1_ref[...],
                                  preferred_element_type=f32) + fb1_ref[...]) + hh
        ec = jax.lax.dot(h2, fw2_ref[...], preferred_element_type=f32)
        es.append(ec + fb2_ref[0])
    ecat = jnp.concatenate(es, axis=1)
    at = atype_ref[...]
    bias = jnp.zeros_like(ecat)
    for t in range(NTYPES):
        bias = jnp.where(at == t, bias_ref[t], bias)
    out_ref[...] = ecat + bias


def _tc_compute(dx, dy, dz, atype4, xl4, yl4, zl4, w0, b0, w1, b1,
                fw0, fb0, fw1, fb1, fw2, fb2, bias):
    rows_all = NPAIR // 128
    vspec = lambda: pl.BlockSpec((P_ROWS, 128), lambda b: (b, 0))
    aspec = lambda: pl.BlockSpec((P_ROWS, 4), lambda b: (b, 0))
    smem = lambda s: pl.BlockSpec(memory_space=pltpu.SMEM)
    full = lambda s: pl.BlockSpec(s, lambda b: tuple(0 for _ in s))
    return pl.pallas_call(
        _tc_body,
        grid=(N_BLOCKS,),
        in_specs=[vspec(), vspec(), vspec(), aspec(),
                  aspec(), aspec(), aspec(),
                  smem((1, 8)), smem((8,)), smem((8, 16)), smem((16,)),
                  full((64, 32)), full((1, 32)), full((32, 32)),
                  full((1, 32)), full((32, 1)),
                  smem((1,)), smem((4,))],
        out_specs=pl.BlockSpec((P_ROWS, 4), lambda b: (b, 0)),
        out_shape=jax.ShapeDtypeStruct((NLOC_PAD // 4, 4), jnp.float32),
    )(dx.reshape(rows_all, 128), dy.reshape(rows_all, 128),
      dz.reshape(rows_all, 128), atype4, xl4, yl4, zl4,
      w0, b0, w1, b1, fw0, fb0.reshape(1, 32), fw1, fb1.reshape(1, 32),
      fw2, fb2, bias)


@jax.jit
def kernel(extended_coord, extended_atype, nlist, w0, b0, w1, b1,
           fw0, fb0, fw1, fb1, fw2, fb2, bias_atom_e):
    coord4 = jnp.pad(extended_coord[0].astype(jnp.float32),
                     ((0, 0), (0, 5)))
    nflat = nlist[0].astype(jnp.int32).reshape(NLOC * NSEL)
    nflat = jnp.pad(nflat, (0, NPAIR - NLOC * NSEL))
    dx, dy, dz = _sc_gather_fn()(coord4, nflat)
    cl = extended_coord[0, :NLOC_PAD].astype(jnp.float32)
    xl4 = cl[:, 0].reshape(NLOC_PAD // 4, 4)
    yl4 = cl[:, 1].reshape(NLOC_PAD // 4, 4)
    zl4 = cl[:, 2].reshape(NLOC_PAD // 4, 4)
    atype4 = jnp.pad(extended_atype[0, :NLOC].astype(jnp.int32),
                     (0, NLOC_PAD - NLOC)).reshape(NLOC_PAD // 4, 4)
    e4 = _tc_compute(dx, dy, dz, atype4, xl4, yl4, zl4, w0, b0, w1, b1,
                     fw0, fb0, fw1, fb1, fw2, fb2, bias_atom_e)
    return e4.reshape(NLOC_PAD)[:NLOC].reshape(1, NLOC)


# speedup vs baseline: 1.5656x; 1.0612x over previous
import functools

import jax
import jax.numpy as jnp
from jax import lax
from jax.experimental import pallas as pl
from jax.experimental.pallas import tpu as pltpu
from jax.experimental.pallas import tpu_sc as plsc

NLOC, NALL, NSEL = 50000, 60000, 32
NTYPES = 4
M1, M2 = 16, 4
RMIN, RMAX = 0.5, 6.0

NLOC_PAD = 51200
NPAIR = NLOC_PAD * NSEL
NWORK = 32
PW = NPAIR // NWORK
CHUNK = 1024
NCHUNK = PW // CHUNK
A_BLK = 3200
P_ROWS = A_BLK * NSEL // 128
N_BLOCKS = NLOC_PAD // A_BLK


NBUF = 3
C0_CHUNKS = 50


def _sc_gather_body(coord_hbm, nlist_hbm, dx_hbm, dy_hbm, dz_hbm, *refs):
    bufs = [refs[5 * b:5 * b + 5] for b in range(NBUF)]
    sems = refs[5 * NBUF:]
    idx_sems = sems[0:NBUF]
    in_sems = sems[NBUF:2 * NBUF]
    out_sems = sems[2 * NBUF:3 * NBUF]
    cid = lax.axis_index("c")
    sid = lax.axis_index("s")
    lane = lax.iota(jnp.int32, 16)
    sub_chunks = 2 * NCHUNK
    c0_base = sid * sub_chunks * CHUNK
    my_base = c0_base + cid * (C0_CHUNKS * CHUNK)

    def chunk_base(k):
        return pl.multiple_of(my_base + k * CHUNK, 256)

    def start_idx(k):
        b = k % NBUF
        return pltpu.async_copy(nlist_hbm.at[pl.ds(chunk_base(k), CHUNK)],
                                bufs[b][0], idx_sems[b])

    def start_gather(k):
        b = k % NBUF
        return pltpu.async_copy(coord_hbm.at[bufs[b][0]], bufs[b][1],
                                in_sems[b])

    def start_outputs(k):
        b = k % NBUF
        _, _, dxv, dyv, dzv = bufs[b]
        base = chunk_base(k)
        return [pltpu.async_copy(dxv, dx_hbm.at[pl.ds(base, CHUNK)], out_sems[b]),
                pltpu.async_copy(dyv, dy_hbm.at[pl.ds(base, CHUNK)], out_sems[b]),
                pltpu.async_copy(dzv, dz_hbm.at[pl.ds(base, CHUNK)], out_sems[b])]

    def compute(k):
        b = k % NBUF
        _, rows, dxv, dyv, dzv = bufs[b]

        def body(t, _):
            p0 = t * 16
            pv = lane + p0
            sl = pl.ds(p0, 16)
            col0 = jnp.full((16,), 0, jnp.int32)
            col1 = jnp.full((16,), 1, jnp.int32)
            col2 = jnp.full((16,), 2, jnp.int32)
            dxv[sl] = plsc.load_gather(rows, [pv, col0])
            dyv[sl] = plsc.load_gather(rows, [pv, col1])
            dzv[sl] = plsc.load_gather(rows, [pv, col2])
            return _

        lax.fori_loop(0, CHUNK // 16, body, None)

    def pipeline(nchunks):
        start_idx(0).wait()
        gat = {0: start_gather(0)}
        idxp = {1: start_idx(1)} if nchunks > 1 else {}
        outs = {}
        for k in range(nchunks):
            if k + 1 < nchunks:
                idxp.pop(k + 1).wait()
                gat[k + 1] = start_gather(k + 1)
            if k + 2 < nchunks:
                idxp[k + 2] = start_idx(k + 2)
            gat.pop(k).wait()
            if k - NBUF + 1 in outs:
                for c in outs.pop(k - NBUF + 1):
                    c.wait()
            compute(k)
            outs[k] = start_outputs(k)
        for cs in outs.values():
            for c in cs:
                c.wait()

    @pl.when(cid == 0)
    def _():
        pipeline(C0_CHUNKS)

    @pl.when(cid == 1)
    def _():
        pipeline(2 * NCHUNK - C0_CHUNKS)


@functools.lru_cache(maxsize=1)
def _sc_gather_fn():
    bufset = [
        pltpu.VMEM((CHUNK,), jnp.int32),
        pltpu.VMEM((CHUNK, 8), jnp.float32),
        pltpu.VMEM((CHUNK,), jnp.float32),
        pltpu.VMEM((CHUNK,), jnp.float32),
        pltpu.VMEM((CHUNK,), jnp.float32),
    ]
    return pl.kernel(
        _sc_gather_body,
        out_type=(jax.ShapeDtypeStruct((NPAIR,), jnp.float32),) * 3,
        mesh=plsc.VectorSubcoreMesh(core_axis_name="c", subcore_axis_name="s"),
        compiler_params=pltpu.CompilerParams(needs_layout_passes=False,
                                             use_tc_tiling_on_sc=False),
        scratch_types=bufset * NBUF + [pltpu.SemaphoreType.DMA] * (3 * NBUF),
    )


def _iota2(shape, axis):
    return lax.broadcasted_iota(jnp.int32, shape, axis)


def _tc_body(dx_ref, dy_ref, dz_ref, atype_ref, xl_ref, yl_ref, zl_ref,
             w0_ref, b0_ref, w1_ref, b1_ref,
             fw0_ref, fb0_ref, fw1_ref, fb1_ref, fw2_ref,
             fb2_ref, bias_ref, out_ref):
    f32 = jnp.float32
    lgrp = _iota2((P_ROWS, 128), 1) // 32

    def _expand(ref):
        a = ref[...]
        xi = jnp.broadcast_to(a[:, 0:1], (P_ROWS, 128))
        for c in range(1, 4):
            xi = jnp.where(lgrp == c,
                           jnp.broadcast_to(a[:, c:c + 1], (P_ROWS, 128)), xi)
        return xi

    dx = dx_ref[...] - _expand(xl_ref)
    dy = dy_ref[...] - _expand(yl_ref)
    dz = dz_ref[...] - _expand(zl_ref)
    r2 = dx * dx + dy * dy + dz * dz + 1e-12
    r = jnp.sqrt(r2)
    rinv = 1.0 / r
    uu = jnp.clip((r - RMIN) / (RMAX - RMIN), 0.0, 1.0)
    uu2 = uu * uu
    sw = uu2 * uu * (-6.0 * uu2 + 15.0 * uu - 10.0) + 1.0
    sr = sw * rinv
    h = [jnp.tanh(sr * w0_ref[0, k] + b0_ref[k]) for k in range(8)]
    g = []
    for i in range(M1):
        acc = jnp.full_like(sr, b1_ref[i])
        for k in range(8):
            acc = acc + h[k] * w1_ref[k, i]
        g.append(jnp.tanh(acc))
    srr = sr * rinv
    R = [sr, srr * dx, srr * dy, srr * dz]

    rr = _iota2((4 * 128, M2 * 4), 0)
    cc = _iota2((4 * 128, M2 * 4), 1)
    bd512 = jnp.where((cc // 4 == rr // 128) & (cc % 4 == (rr % 128) // 32),
                      f32(1.0 / NSEL), f32(0.0))
    parts = []
    for i in range(M1):
        pi = jnp.concatenate([g[i] * R[j] for j in range(M2)], axis=1)
        parts.append(jax.lax.dot(pi, bd512, preferred_element_type=f32))
    grcat = jnp.concatenate(parts, axis=1)

    k4r = _iota2((M1 * M2, M1), 0)
    k4c = _iota2((M1 * M2, M1), 1)
    k4 = jnp.where(k4r // 4 == k4c, f32(1.0), f32(0.0))
    es = []
    for c in range(4):
        sr_ = _iota2((256, 64), 0)
        sc_ = _iota2((256, 64), 1)
        sel = jnp.where((sr_ % 4 == c) & (sr_ // 16 == sc_ // 4)
                        & ((sr_ % 16) // 4 == sc_ % 4), f32(1.0), f32(0.0))
        grc = jax.lax.dot(grcat, sel, preferred_element_type=f32)
        d = None
        for k in range(M2):
            br = _iota2((64, 64), 0)
            bc = _iota2((64, 64), 1)
            bk = jnp.where((br // 4 == k) & (br % 4 == bc % 4),
                           f32(1.0), f32(0.0))
            tk = jax.lax.dot(grc, bk, preferred_element_type=f32)
            dk = jax.lax.dot(grc * tk, k4, preferred_element_type=f32)
            skr = _iota2((M1, 64), 0)
            skc = _iota2((M1, 64), 1)
            s_k = jnp.where(skc == 4 * skr + k, f32(1.0), f32(0.0))
            dsc = jax.lax.dot(dk, s_k, preferred_element_type=f32)
            d = dsc if d is None else d + dsc
        hh = jnp.tanh(jax.lax.dot(d, fw0_ref[...],
                                  preferred_element_type=f32) + fb0_ref[...])
        h2 = jnp.tanh(jax.lax.dot(hh, fw1_ref[...],
                                  preferred_element_type=f32) + fb1_ref[...]) + hh
        ec = jax.lax.dot(h2, fw2_ref[...], preferred_element_type=f32)
        es.append(ec + fb2_ref[0])
    ecat = jnp.concatenate(es, axis=1)
    at = atype_ref[...]
    bias = jnp.zeros_like(ecat)
    for t in range(NTYPES):
        bias = jnp.where(at == t, bias_ref[t], bias)
    out_ref[...] = ecat + bias


def _tc_compute(dx, dy, dz, atype4, xl4, yl4, zl4, w0, b0, w1, b1,
                fw0, fb0, fw1, fb1, fw2, fb2, bias):
    rows_all = NPAIR // 128
    vspec = lambda: pl.BlockSpec((P_ROWS, 128), lambda b: (b, 0))
    aspec = lambda: pl.BlockSpec((P_ROWS, 4), lambda b: (b, 0))
    smem = lambda s: pl.BlockSpec(memory_space=pltpu.SMEM)
    full = lambda s: pl.BlockSpec(s, lambda b: tuple(0 for _ in s))
    return pl.pallas_call(
        _tc_body,
        grid=(N_BLOCKS,),
        in_specs=[vspec(), vspec(), vspec(), aspec(),
                  aspec(), aspec(), aspec(),
                  smem((1, 8)), smem((8,)), smem((8, 16)), smem((16,)),
                  full((64, 32)), full((1, 32)), full((32, 32)),
                  full((1, 32)), full((32, 1)),
                  smem((1,)), smem((4,))],
        out_specs=pl.BlockSpec((P_ROWS, 4), lambda b: (b, 0)),
        out_shape=jax.ShapeDtypeStruct((NLOC_PAD // 4, 4), jnp.float32),
    )(dx.reshape(rows_all, 128), dy.reshape(rows_all, 128),
      dz.reshape(rows_all, 128), atype4, xl4, yl4, zl4,
      w0, b0, w1, b1, fw0, fb0.reshape(1, 32), fw1, fb1.reshape(1, 32),
      fw2, fb2, bias)


@jax.jit
def kernel(extended_coord, extended_atype, nlist, w0, b0, w1, b1,
           fw0, fb0, fw1, fb1, fw2, fb2, bias_atom_e):
    coord4 = jnp.pad(extended_coord[0].astype(jnp.float32),
                     ((0, 0), (0, 5)))
    nflat = nlist[0].astype(jnp.int32).reshape(NLOC * NSEL)
    nflat = jnp.pad(nflat, (0, NPAIR - NLOC * NSEL))
    dx, dy, dz = _sc_gather_fn()(coord4, nflat)
    cl = extended_coord[0, :NLOC_PAD].astype(jnp.float32)
    xl4 = cl[:, 0].reshape(NLOC_PAD // 4, 4)
    yl4 = cl[:, 1].reshape(NLOC_PAD // 4, 4)
    zl4 = cl[:, 2].reshape(NLOC_PAD // 4, 4)
    atype4 = jnp.pad(extended_atype[0, :NLOC].astype(jnp.int32),
                     (0, NLOC_PAD - NLOC)).reshape(NLOC_PAD // 4, 4)
    e4 = _tc_compute(dx, dy, dz, atype4, xl4, yl4, zl4, w0, b0, w1, b1,
                     fw0, fb0, fw1, fb1, fw2, fb2, bias_atom_e)
    return e4.reshape(NLOC_PAD)[:NLOC].reshape(1, NLOC)
